# baseline (speedup 1.0000x reference)
import functools

import jax
import jax.numpy as jnp
from jax import lax
from jax.experimental import pallas as pl
from jax.experimental.pallas import tpu as pltpu
from jax.experimental.pallas import tpu_sc as plsc

_N = 10000
_R = 320
_G = _R // 16
_NC = 2
_NS = 16
_MAGIC = 0x5F3759DF


def _iota16():
    return lax.broadcasted_iota(jnp.int32, (16,), 0)


def _compute_side(rand_v, tail_v, out_v):
    i16 = _iota16()
    c13 = jnp.full((16,), 13, jnp.int32)
    c15 = jnp.full((16,), 15, jnp.int32)

    def group(g, carry):
        sl = pl.ds(g * 16, 16)
        l0 = rand_v[0, sl]
        l1 = rand_v[1, sl]
        l2 = rand_v[2, sl]
        s = l0 * l0 + l1 * l1 + l2 * l2
        s = jnp.maximum(s, 1e-24)
        i = plsc.bitcast(s, jnp.int32)
        i = _MAGIC - lax.shift_right_logical(i, 1)
        y = plsc.bitcast(i, jnp.float32)
        for _ in range(3):
            y = y * (1.5 - 0.5 * s * y * y)
        scale = 10.0 * y
        ridx = i16 + (g * 16)
        a = plsc.load_gather(tail_v, [ridx, c13])
        b = plsc.load_gather(tail_v, [ridx, c15])
        out_v[0, sl] = l0 * scale + jnp.where(a != 0.0, -10.0, 0.0)
        out_v[1, sl] = l1 * scale
        out_v[2, sl] = l2 * scale + jnp.where(b != 0.0, -10.0, 0.0)
        return carry

    lax.fori_loop(0, _G, group, 0, unroll=2)


def _body(xs, xt, lr, rr, left_o, right_o,
          tail_l, tail_r, rand_l, rand_r, out_l, out_r,
          sem_l, sem_r, sem_rl, sem_rr):
    cid = lax.axis_index("c")
    sid = lax.axis_index("s")
    wid = sid * _NC + cid
    base = jnp.minimum(wid * _R, _N - _R)

    cl = pltpu.async_copy(xs.at[pl.ds(base, _R), pl.ds(112, 16)], tail_l, sem_l)
    cr = pltpu.async_copy(xt.at[pl.ds(base, _R), pl.ds(112, 16)], tail_r, sem_r)
    crl = pltpu.async_copy(lr.at[:, pl.ds(base, _R)], rand_l, sem_rl)
    crr = pltpu.async_copy(rr.at[:, pl.ds(base, _R)], rand_r, sem_rr)

    crl.wait()
    cl.wait()
    _compute_side(rand_l, tail_l, out_l)
    pltpu.sync_copy(out_l, left_o.at[:, pl.ds(base, _R)])

    crr.wait()
    cr.wait()
    _compute_side(rand_r, tail_r, out_r)
    pltpu.sync_copy(out_r, right_o.at[:, pl.ds(base, _R)])


@jax.jit
def _run(xs, xt, lr_t, rr_t):
    f32 = jnp.float32
    k = functools.partial(
        pl.kernel,
        out_type=(jax.ShapeDtypeStruct((3, _N), f32),
                  jax.ShapeDtypeStruct((3, _N), f32)),
        mesh=plsc.VectorSubcoreMesh(core_axis_name="c", subcore_axis_name="s"),
        compiler_params=pltpu.CompilerParams(
            needs_layout_passes=False, use_tc_tiling_on_sc=False),
        scratch_types=[
            pltpu.VMEM((_R, 16), f32),
            pltpu.VMEM((_R, 16), f32),
            pltpu.VMEM((3, _R), f32),
            pltpu.VMEM((3, _R), f32),
            pltpu.VMEM((3, _R), f32),
            pltpu.VMEM((3, _R), f32),
            pltpu.SemaphoreType.DMA,
            pltpu.SemaphoreType.DMA,
            pltpu.SemaphoreType.DMA,
            pltpu.SemaphoreType.DMA,
        ],
    )(_body)
    return k(xs, xt, lr_t, rr_t)


def kernel(x_s, x_t, edge_index, left_rand, right_rand):
    del edge_index
    left_t, right_t = _run(x_s, x_t, left_rand.T, right_rand.T)
    return left_t.T, right_t.T


# speedup vs baseline: 1.0242x; 1.0242x over previous
import functools

import jax
import jax.numpy as jnp
from jax import lax
from jax.experimental import pallas as pl
from jax.experimental.pallas import tpu as pltpu
from jax.experimental.pallas import tpu_sc as plsc

_N = 10000
_R = 320
_G = _R // 16
_NC = 2
_NS = 16
_MAGIC = 0x5F3759DF


def _iota16():
    return lax.broadcasted_iota(jnp.int32, (16,), 0)


def _compute_side(rand_v, tail_v, out_v):
    i16 = _iota16()
    c13 = jnp.full((16,), 13, jnp.int32)
    c15 = jnp.full((16,), 15, jnp.int32)

    @plsc.parallel_loop(0, _R, 16, unroll=4)
    def group(r):
        sl = pl.ds(r, 16)
        l0 = rand_v[0, sl]
        l1 = rand_v[1, sl]
        l2 = rand_v[2, sl]
        s = l0 * l0 + l1 * l1 + l2 * l2
        s = jnp.maximum(s, 1e-24)
        i = plsc.bitcast(s, jnp.int32)
        i = _MAGIC - lax.shift_right_logical(i, 1)
        y = plsc.bitcast(i, jnp.float32)
        for _ in range(2):
            y = y * (1.5 - 0.5 * s * y * y)
        scale = 10.0 * y
        ridx = i16 + r
        a = plsc.load_gather(tail_v, [ridx, c13])
        b = plsc.load_gather(tail_v, [ridx, c15])
        out_v[0, sl] = l0 * scale + jnp.where(a != 0.0, -10.0, 0.0)
        out_v[1, sl] = l1 * scale
        out_v[2, sl] = l2 * scale + jnp.where(b != 0.0, -10.0, 0.0)


def _body(xs, xt, lr, rr, left_o, right_o,
          tail_l, tail_r, rand_l, rand_r, out_l, out_r,
          sem_l, sem_r, sem_rl, sem_rr):
    cid = lax.axis_index("c")
    sid = lax.axis_index("s")
    wid = sid * _NC + cid
    base = jnp.minimum(wid * _R, _N - _R)

    cl = pltpu.async_copy(xs.at[pl.ds(base, _R), pl.ds(112, 16)], tail_l, sem_l)
    cr = pltpu.async_copy(xt.at[pl.ds(base, _R), pl.ds(112, 16)], tail_r, sem_r)
    crl = pltpu.async_copy(lr.at[:, pl.ds(base, _R)], rand_l, sem_rl)
    crr = pltpu.async_copy(rr.at[:, pl.ds(base, _R)], rand_r, sem_rr)

    crl.wait()
    cl.wait()
    _compute_side(rand_l, tail_l, out_l)
    pltpu.sync_copy(out_l, left_o.at[:, pl.ds(base, _R)])

    crr.wait()
    cr.wait()
    _compute_side(rand_r, tail_r, out_r)
    pltpu.sync_copy(out_r, right_o.at[:, pl.ds(base, _R)])


@jax.jit
def _run(xs, xt, lr_t, rr_t):
    f32 = jnp.float32
    k = functools.partial(
        pl.kernel,
        out_type=(jax.ShapeDtypeStruct((3, _N), f32),
                  jax.ShapeDtypeStruct((3, _N), f32)),
        mesh=plsc.VectorSubcoreMesh(core_axis_name="c", subcore_axis_name="s"),
        compiler_params=pltpu.CompilerParams(
            needs_layout_passes=False, use_tc_tiling_on_sc=False),
        scratch_types=[
            pltpu.VMEM((_R, 16), f32),
            pltpu.VMEM((_R, 16), f32),
            pltpu.VMEM((3, _R), f32),
            pltpu.VMEM((3, _R), f32),
            pltpu.VMEM((3, _R), f32),
            pltpu.VMEM((3, _R), f32),
            pltpu.SemaphoreType.DMA,
            pltpu.SemaphoreType.DMA,
            pltpu.SemaphoreType.DMA,
            pltpu.SemaphoreType.DMA,
        ],
    )(_body)
    return k(xs, xt, lr_t, rr_t)


def kernel(x_s, x_t, edge_index, left_rand, right_rand):
    del edge_index
    left_t, right_t = _run(x_s, x_t, left_rand.T, right_rand.T)
    return left_t.T, right_t.T
